# Initial kernel scaffold; baseline (speedup 1.0000x reference)
import jax
import jax.numpy as jnp
from jax.experimental import pallas as pl


def kernel(x_real, x_fake):
    raise NotImplementedError("write your pallas kernel here")


# speedup vs baseline: 40.8822x; 40.8822x over previous
import functools

import jax
import jax.numpy as jnp
from jax import lax
from jax.experimental import pallas as pl
from jax.experimental.pallas import tpu as pltpu
from jax.experimental.pallas import tpu_sc as plsc

NC, NS, L = 2, 16, 16
NW = NC * NS
B, C, H, W = 32, 3, 512, 512
PLANE = H * W
CHUNK = 8192
NCHUNK = PLANE // CHUNK
GROUPS = CHUNK // L
NBINS = 10
ROWS = 6 * NBINS

_mesh = plsc.VectorSubcoreMesh(
    core_axis_name="c", subcore_axis_name="s", num_cores=NC, num_subcores=NS
)


@functools.partial(
    pl.kernel,
    out_type=jax.ShapeDtypeStruct((NW, ROWS * L), jnp.float32),
    mesh=_mesh,
    compiler_params=pltpu.CompilerParams(needs_layout_passes=False),
    scratch_types=[
        pltpu.VMEM((CHUNK,), jnp.float32),
        pltpu.VMEM((CHUNK,), jnp.float32),
        pltpu.VMEM((CHUNK,), jnp.float32),
        pltpu.VMEM((ROWS * L,), jnp.float32),
    ],
)
def _hist_kernel(real_hbm, fake_hbm, out_hbm, rbuf, gbuf, bbuf, hist):
    wid = lax.axis_index("s") * NC + lax.axis_index("c")
    zero = jnp.zeros((L,), jnp.float32)
    for i in range(ROWS):
        hist[pl.ds(i * L, L)] = zero
    lanes = lax.iota(jnp.int32, L)
    ones = jnp.ones((L,), jnp.float32)

    def do_image(src_hbm, base_row):
        plane0 = wid * (C * PLANE)

        def chunk_body(k, carry):
            off = plane0 + k * CHUNK
            pltpu.sync_copy(src_hbm.at[pl.ds(off, CHUNK)], rbuf)
            pltpu.sync_copy(src_hbm.at[pl.ds(off + PLANE, CHUNK)], gbuf)
            pltpu.sync_copy(src_hbm.at[pl.ds(off + 2 * PLANE, CHUNK)], bbuf)

            def grp(i, c2):
                s = i * L
                r = jnp.clip(rbuf[pl.ds(s, L)], 0.0, 1.0)
                g = jnp.clip(gbuf[pl.ds(s, L)], 0.0, 1.0)
                b = jnp.clip(bbuf[pl.ds(s, L)], 0.0, 1.0)
                mx = jnp.maximum(r, jnp.maximum(g, b))
                mn = jnp.minimum(r, jnp.minimum(g, b))
                d = mx - mn
                nz = d != 0.0
                safe = jnp.where(nz, d, 1.0)
                q = (g - b) / safe
                hr = jnp.where(q < 0.0, q + 6.0, q)
                hg = (b - r) / safe + 2.0
                hb = (r - g) / safe + 4.0
                mask_r = (mx == r) & nz
                mask_g = (mx == g) & nz
                mask_b = (mx == b) & nz
                hue = jnp.where(mask_b, hb, jnp.where(mask_g, hg, jnp.where(mask_r, hr, 0.0)))
                hue = hue / 6.0
                mxnz = mx != 0.0
                safe_mx = jnp.where(mxnz, mx, 1.0)
                sat = jnp.where(mxnz, d / safe_mx, 0.0)
                bh = jnp.minimum((hue * 10.0).astype(jnp.int32), 9)
                bs = jnp.minimum((sat * 10.0).astype(jnp.int32), 9)
                bv = jnp.minimum((mx * 10.0).astype(jnp.int32), 9)
                base = base_row * L + lanes
                plsc.addupdate_scatter(hist, [bh * L + base], ones)
                plsc.addupdate_scatter(hist, [bs * L + (base + NBINS * L)], ones)
                plsc.addupdate_scatter(hist, [bv * L + (base + 2 * NBINS * L)], ones)
                return c2

            lax.fori_loop(0, GROUPS, grp, 0)
            return carry

        lax.fori_loop(0, NCHUNK, chunk_body, 0)

    do_image(real_hbm, 0)
    do_image(fake_hbm, 3 * NBINS)
    pltpu.sync_copy(hist, out_hbm.at[wid])


def _loss_body(hist_ref, out_ref):
    h = hist_ref[...]
    t = jnp.sum(h, axis=0)
    tot = jnp.sum(t, axis=1, keepdims=True)
    d = jnp.abs(tot[: 3 * NBINS] - tot[3 * NBINS :])
    w = jnp.concatenate(
        [
            jnp.full((NBINS, 1), 0.3 / NBINS, jnp.float32),
            jnp.full((NBINS, 1), 0.4 / NBINS, jnp.float32),
            jnp.full((NBINS, 1), 0.4 / NBINS, jnp.float32),
        ],
        axis=0,
    )
    out_ref[0, 0] = jnp.sum(d * w)


def kernel(x_real, x_fake):
    part = _hist_kernel(x_real.reshape(-1), x_fake.reshape(-1))
    part = part.reshape(NW, ROWS, L)
    loss = pl.pallas_call(
        _loss_body,
        out_shape=jax.ShapeDtypeStruct((1, 1), jnp.float32),
        out_specs=pl.BlockSpec(memory_space=pltpu.SMEM),
    )(part)
    return loss[0, 0]
